# baseline (device time: 127371 ns/iter reference)
import jax
import jax.numpy as jnp
from jax import lax
from jax.experimental import pallas as pl
from jax.experimental.pallas import tpu as pltpu

M_PER = 8192
N = 1024
HALF = M_PER // 2
K = 32
CH = HALF // K


def kernel(x):
    def body(
        x_hbm,
        out_hbm,
        stage_in,
        stage_bf,
        rx1,
        rx2,
        in_sems,
        st_sems,
        send1,
        recv1,
        send2,
        recv2,
    ):
        my_x = lax.axis_index("x")
        my_y = lax.axis_index("y")
        x_peer = (1 - my_x, my_y)
        y_peer = (my_x, 1 - my_y)

        barrier_sem = pltpu.get_barrier_semaphore()
        for nbr in (x_peer, y_peer):
            pl.semaphore_signal(
                barrier_sem, inc=1,
                device_id=nbr, device_id_type=pl.DeviceIdType.MESH,
            )
        pl.semaphore_wait(barrier_sem, 2)

        my_base = my_x * M_PER
        my_half_r0 = my_y * HALF
        oth_half_r0 = (1 - my_y) * HALF
        base_send1 = my_base + my_half_r0
        base_fwd = (1 - my_x) * M_PER + my_half_r0
        base_recv2 = (1 - my_x) * M_PER + oth_half_r0

        in_dmas = []
        for j in range(2 * K):
            r0 = my_half_r0 if j < K else oth_half_r0
            cc = j % K
            d = pltpu.make_async_copy(
                x_hbm.at[pl.ds(r0 + cc * CH, CH), :],
                stage_in.at[pl.ds(cc * CH, CH), :],
                in_sems.at[j],
            )
            if j < K:
                d.start()
            in_dmas.append(d)

        st_dmas = []

        send1_descs = []
        for c in range(K):
            rows = pl.ds(my_half_r0 + c * CH, CH)
            slot = pl.ds(c * CH, CH)
            in_dmas[c].wait()
            stage_bf[rows, :] = stage_in[slot, :].astype(stage_bf.dtype)
            d = pltpu.make_async_remote_copy(
                src_ref=stage_bf.at[rows, :],
                dst_ref=rx1.at[slot, :],
                send_sem=send1.at[c],
                recv_sem=recv1.at[c],
                device_id=x_peer,
                device_id_type=pl.DeviceIdType.MESH,
            )
            d.start()
            send1_descs.append(d)
            st = pltpu.make_async_copy(
                stage_bf.at[rows, :],
                out_hbm.at[pl.ds(base_send1 + c * CH, CH), :],
                st_sems.at[c],
            )
            st.start()
            st_dmas.append(st)
            in_dmas[K + c].start()

        send2_descs = []
        recv2_descs = []
        for c in range(K):
            slot = pl.ds(c * CH, CH)
            recv1_d = pltpu.make_async_remote_copy(
                src_ref=rx1.at[slot, :],
                dst_ref=rx1.at[slot, :],
                send_sem=send1.at[c],
                recv_sem=recv1.at[c],
                device_id=x_peer,
                device_id_type=pl.DeviceIdType.MESH,
            )
            recv1_d.wait_recv()
            d = pltpu.make_async_remote_copy(
                src_ref=rx1.at[slot, :],
                dst_ref=rx2.at[slot, :],
                send_sem=send2.at[c],
                recv_sem=recv2.at[c],
                device_id=y_peer,
                device_id_type=pl.DeviceIdType.MESH,
            )
            d.start()
            send2_descs.append(d)
            st = pltpu.make_async_copy(
                rx1.at[slot, :],
                out_hbm.at[pl.ds(base_fwd + c * CH, CH), :],
                st_sems.at[2 * K + c],
            )
            st.start()
            st_dmas.append(st)
            recv2_descs.append(
                pltpu.make_async_remote_copy(
                    src_ref=rx2.at[slot, :],
                    dst_ref=rx2.at[slot, :],
                    send_sem=send2.at[c],
                    recv_sem=recv2.at[c],
                    device_id=y_peer,
                    device_id_type=pl.DeviceIdType.MESH,
                )
            )
            rows = pl.ds(oth_half_r0 + c * CH, CH)
            in_dmas[K + c].wait()
            stage_bf[rows, :] = stage_in[slot, :].astype(stage_bf.dtype)
            st = pltpu.make_async_copy(
                stage_bf.at[rows, :],
                out_hbm.at[pl.ds(my_base + oth_half_r0 + c * CH, CH), :],
                st_sems.at[K + c],
            )
            st.start()
            st_dmas.append(st)

        for c in range(K):
            slot = pl.ds(c * CH, CH)
            recv2_descs[c].wait_recv()
            st = pltpu.make_async_copy(
                rx2.at[slot, :],
                out_hbm.at[pl.ds(base_recv2 + c * CH, CH), :],
                st_sems.at[3 * K + c],
            )
            st.start()
            st_dmas.append(st)

        for st in st_dmas:
            st.wait()
        for d in send1_descs:
            d.wait_send()
        for d in send2_descs:
            d.wait_send()

    out_shape = jax.ShapeDtypeStruct((2 * M_PER, N), jnp.bfloat16)
    return pl.pallas_call(
        body,
        out_shape=out_shape,
        in_specs=[pl.BlockSpec(memory_space=pl.ANY)],
        out_specs=pl.BlockSpec(memory_space=pl.ANY),
        scratch_shapes=[
            pltpu.VMEM((HALF, N), jnp.float32),
            pltpu.VMEM((M_PER, N), jnp.bfloat16),
            pltpu.VMEM((HALF, N), jnp.bfloat16),
            pltpu.VMEM((HALF, N), jnp.bfloat16),
            pltpu.SemaphoreType.DMA((2 * K,)),
            pltpu.SemaphoreType.DMA((4 * K,)),
            pltpu.SemaphoreType.DMA((K,)),
            pltpu.SemaphoreType.DMA((K,)),
            pltpu.SemaphoreType.DMA((K,)),
            pltpu.SemaphoreType.DMA((K,)),
        ],
        compiler_params=pltpu.CompilerParams(
            collective_id=0, vmem_limit_bytes=64 * 1024 * 1024
        ),
    )(x)


# device time: 126200 ns/iter; 1.0093x vs baseline; 1.0093x over previous
import jax
import jax.numpy as jnp
from jax import lax
from jax.experimental import pallas as pl
from jax.experimental.pallas import tpu as pltpu

M_PER = 8192
N = 1024
HALF = M_PER // 2
K = 32
CH = HALF // K


def kernel(x):
    def body(
        x_hbm,
        out_hbm,
        stage_in,
        stage_bf,
        rx1,
        rx2,
        in_sems,
        st_sems,
        send1,
        recv1,
        send2,
        recv2,
    ):
        my_x = lax.axis_index("x")
        my_y = lax.axis_index("y")
        x_peer = (1 - my_x, my_y)
        y_peer = (my_x, 1 - my_y)

        my_base = my_x * M_PER
        my_half_r0 = my_y * HALF
        oth_half_r0 = (1 - my_y) * HALF
        base_send1 = my_base + my_half_r0
        base_fwd = (1 - my_x) * M_PER + my_half_r0
        base_recv2 = (1 - my_x) * M_PER + oth_half_r0

        in_dmas = []
        for j in range(2 * K):
            r0 = my_half_r0 if j < K else oth_half_r0
            cc = j % K
            d = pltpu.make_async_copy(
                x_hbm.at[pl.ds(r0 + cc * CH, CH), :],
                stage_in.at[pl.ds(cc * CH, CH), :],
                in_sems.at[j],
            )
            if j < K:
                d.start()
            in_dmas.append(d)

        barrier_sem = pltpu.get_barrier_semaphore()
        for nbr in (x_peer, y_peer):
            pl.semaphore_signal(
                barrier_sem, inc=1,
                device_id=nbr, device_id_type=pl.DeviceIdType.MESH,
            )
        pl.semaphore_wait(barrier_sem, 2)

        st_dmas = []

        send1_descs = []
        for c in range(K):
            rows = pl.ds(my_half_r0 + c * CH, CH)
            slot = pl.ds(c * CH, CH)
            in_dmas[c].wait()
            stage_bf[rows, :] = stage_in[slot, :].astype(stage_bf.dtype)
            d = pltpu.make_async_remote_copy(
                src_ref=stage_bf.at[rows, :],
                dst_ref=rx1.at[slot, :],
                send_sem=send1.at[c],
                recv_sem=recv1.at[c],
                device_id=x_peer,
                device_id_type=pl.DeviceIdType.MESH,
            )
            d.start()
            send1_descs.append(d)
            st = pltpu.make_async_copy(
                stage_bf.at[rows, :],
                out_hbm.at[pl.ds(base_send1 + c * CH, CH), :],
                st_sems.at[c],
            )
            st.start()
            st_dmas.append(st)
            in_dmas[K + c].start()

        send2_descs = []
        recv2_descs = []
        for c in range(K):
            slot = pl.ds(c * CH, CH)
            recv1_d = pltpu.make_async_remote_copy(
                src_ref=rx1.at[slot, :],
                dst_ref=rx1.at[slot, :],
                send_sem=send1.at[c],
                recv_sem=recv1.at[c],
                device_id=x_peer,
                device_id_type=pl.DeviceIdType.MESH,
            )
            recv1_d.wait_recv()
            d = pltpu.make_async_remote_copy(
                src_ref=rx1.at[slot, :],
                dst_ref=rx2.at[slot, :],
                send_sem=send2.at[c],
                recv_sem=recv2.at[c],
                device_id=y_peer,
                device_id_type=pl.DeviceIdType.MESH,
            )
            d.start()
            send2_descs.append(d)
            st = pltpu.make_async_copy(
                rx1.at[slot, :],
                out_hbm.at[pl.ds(base_fwd + c * CH, CH), :],
                st_sems.at[2 * K + c],
            )
            st.start()
            st_dmas.append(st)
            recv2_descs.append(
                pltpu.make_async_remote_copy(
                    src_ref=rx2.at[slot, :],
                    dst_ref=rx2.at[slot, :],
                    send_sem=send2.at[c],
                    recv_sem=recv2.at[c],
                    device_id=y_peer,
                    device_id_type=pl.DeviceIdType.MESH,
                )
            )
            rows = pl.ds(oth_half_r0 + c * CH, CH)
            in_dmas[K + c].wait()
            stage_bf[rows, :] = stage_in[slot, :].astype(stage_bf.dtype)
            st = pltpu.make_async_copy(
                stage_bf.at[rows, :],
                out_hbm.at[pl.ds(my_base + oth_half_r0 + c * CH, CH), :],
                st_sems.at[K + c],
            )
            st.start()
            st_dmas.append(st)

        for c in range(K):
            slot = pl.ds(c * CH, CH)
            recv2_descs[c].wait_recv()
            st = pltpu.make_async_copy(
                rx2.at[slot, :],
                out_hbm.at[pl.ds(base_recv2 + c * CH, CH), :],
                st_sems.at[3 * K + c],
            )
            st.start()
            st_dmas.append(st)

        for st in st_dmas:
            st.wait()
        for d in send1_descs:
            d.wait_send()
        for d in send2_descs:
            d.wait_send()

    out_shape = jax.ShapeDtypeStruct((2 * M_PER, N), jnp.bfloat16)
    return pl.pallas_call(
        body,
        out_shape=out_shape,
        in_specs=[pl.BlockSpec(memory_space=pl.ANY)],
        out_specs=pl.BlockSpec(memory_space=pl.ANY),
        scratch_shapes=[
            pltpu.VMEM((HALF, N), jnp.float32),
            pltpu.VMEM((M_PER, N), jnp.bfloat16),
            pltpu.VMEM((HALF, N), jnp.bfloat16),
            pltpu.VMEM((HALF, N), jnp.bfloat16),
            pltpu.SemaphoreType.DMA((2 * K,)),
            pltpu.SemaphoreType.DMA((4 * K,)),
            pltpu.SemaphoreType.DMA((K,)),
            pltpu.SemaphoreType.DMA((K,)),
            pltpu.SemaphoreType.DMA((K,)),
            pltpu.SemaphoreType.DMA((K,)),
        ],
        compiler_params=pltpu.CompilerParams(
            collective_id=0, vmem_limit_bytes=64 * 1024 * 1024
        ),
    )(x)
